# baseline (speedup 1.0000x reference)
import jax
import jax.numpy as jnp
from jax import lax
from jax.experimental import pallas as pl
from jax.experimental.pallas import tpu as pltpu
from jax.experimental.pallas import tpu_sc as plsc

ROWS, FEAT = 52, 5
NI, NJ = 16384, 50
NW = 32
W = 256
SLABS = NI // (NW * W)
TROWS = ROWS
HALVES = ((0, 24), (24, 26))


def _body(idx_hbm, tcols_hbm, out_hbm,
          table_v, idx_v0, idx_v1, out_v,
          in_sem0, in_sem1, out_semA, out_semB):
    idx_bufs = (idx_v0, idx_v1)
    out_sems = (out_semA, out_semB)
    wid = lax.axis_index("c") * 16 + lax.axis_index("s")
    base = wid * SLABS * W
    in_copies = [
        pltpu.async_copy(idx_hbm.at[:, pl.ds(base + s * W, W)],
                         idx_bufs[s], (in_sem0, in_sem1)[s])
        for s in range(SLABS)
    ]
    pltpu.sync_copy(tcols_hbm, table_v)
    fvecs = [jnp.full((16,), f, jnp.int32) for f in range(FEAT)]
    pending = {}
    for s in range(SLABS):
        idx_v = idx_bufs[s]
        in_copies[s].wait()
        for h, (j0, nrows) in enumerate(HALVES):
            if j0 in pending:
                pending[j0].wait()

            @plsc.parallel_loop(j0 * (W // 16), (j0 + nrows) * (W // 16),
                                unroll=4)
            def t_body(t, idx_v=idx_v):
                j = t >> 4
                o = (t & 15) << 4
                vi = idx_v[j, pl.ds(o, 16)]
                for f in range(FEAT):
                    g = plsc.load_gather(table_v, [fvecs[f], vi])
                    out_v[f, j, pl.ds(o, 16)] = g
            pending[j0] = pltpu.async_copy(
                out_v.at[:, pl.ds(j0, nrows), :],
                out_hbm.at[:, pl.ds(j0, nrows), pl.ds(base + s * W, W)],
                out_sems[h])
    for d in pending.values():
        d.wait()


_mesh = plsc.VectorSubcoreMesh(core_axis_name="c", subcore_axis_name="s")

_lookup = pl.kernel(
    _body,
    out_type=jax.ShapeDtypeStruct((FEAT, NJ, NI), jnp.float32),
    mesh=_mesh,
    scratch_types=[
        pltpu.VMEM((FEAT, TROWS), jnp.float32),
        pltpu.VMEM((NJ, W), jnp.int32),
        pltpu.VMEM((NJ, W), jnp.int32),
        pltpu.VMEM((FEAT, NJ, W), jnp.float32),
        pltpu.SemaphoreType.DMA,
        pltpu.SemaphoreType.DMA,
        pltpu.SemaphoreType.DMA,
        pltpu.SemaphoreType.DMA,
    ],
    compiler_params=pltpu.CompilerParams(needs_layout_passes=False),
)


@jax.jit
def kernel(card_indices, table):
    idx_t = card_indices.T
    tcols = table.T
    a = _lookup(idx_t, tcols)
    return jnp.transpose(a, (2, 1, 0))


# speedup vs baseline: 1.0042x; 1.0042x over previous
import jax
import jax.numpy as jnp
from jax import lax
from jax.experimental import pallas as pl
from jax.experimental.pallas import tpu as pltpu
from jax.experimental.pallas import tpu_sc as plsc

ROWS, FEAT = 52, 5
NI, NJ = 16384, 50
NW = 32
W = 256
SLABS = NI // (NW * W)
TROWS = ROWS
HALVES = ((0, 24), (24, 26))


def _body(idx_hbm, tcols_hbm, out_hbm,
          table_v, idx_v0, idx_v1, out_v,
          in_sem0, in_sem1, out_semA, out_semB):
    idx_bufs = (idx_v0, idx_v1)
    out_sems = (out_semA, out_semB)
    wid = lax.axis_index("c") * 16 + lax.axis_index("s")
    base = wid * SLABS * W
    in_copies = [
        pltpu.async_copy(idx_hbm.at[:, pl.ds(base + s * W, W)],
                         idx_bufs[s], (in_sem0, in_sem1)[s])
        for s in range(SLABS)
    ]
    pltpu.sync_copy(tcols_hbm, table_v)
    fvecs = [jnp.full((16,), f, jnp.int32) for f in range(FEAT)]
    pending = {}
    for s in range(SLABS):
        idx_v = idx_bufs[s]
        in_copies[s].wait()
        for h, (j0, nrows) in enumerate(HALVES):
            if j0 in pending:
                pending[j0].wait()

            @plsc.parallel_loop(j0 * (W // 16), (j0 + nrows) * (W // 16),
                                unroll=2)
            def t_body(t, idx_v=idx_v):
                j = t >> 4
                o = (t & 15) << 4
                vi = idx_v[j, pl.ds(o, 16)]
                for f in range(FEAT):
                    g = plsc.load_gather(table_v, [fvecs[f], vi])
                    out_v[f, j, pl.ds(o, 16)] = g
            pending[j0] = pltpu.async_copy(
                out_v.at[:, pl.ds(j0, nrows), :],
                out_hbm.at[:, pl.ds(j0, nrows), pl.ds(base + s * W, W)],
                out_sems[h])
    for d in pending.values():
        d.wait()


_mesh = plsc.VectorSubcoreMesh(core_axis_name="c", subcore_axis_name="s")

_lookup = pl.kernel(
    _body,
    out_type=jax.ShapeDtypeStruct((FEAT, NJ, NI), jnp.float32),
    mesh=_mesh,
    scratch_types=[
        pltpu.VMEM((FEAT, TROWS), jnp.float32),
        pltpu.VMEM((NJ, W), jnp.int32),
        pltpu.VMEM((NJ, W), jnp.int32),
        pltpu.VMEM((FEAT, NJ, W), jnp.float32),
        pltpu.SemaphoreType.DMA,
        pltpu.SemaphoreType.DMA,
        pltpu.SemaphoreType.DMA,
        pltpu.SemaphoreType.DMA,
    ],
    compiler_params=pltpu.CompilerParams(needs_layout_passes=False),
)


@jax.jit
def kernel(card_indices, table):
    idx_t = card_indices.T
    tcols = table.T
    a = _lookup(idx_t, tcols)
    return jnp.transpose(a, (2, 1, 0))
